# Initial kernel scaffold; baseline (speedup 1.0000x reference)
import jax
import jax.numpy as jnp
from jax.experimental import pallas as pl


def kernel(context_vars, tables, W1, b1, W2, b2, Wh, bh):
    raise NotImplementedError("write your pallas kernel here")


# speedup vs baseline: 2.2432x; 2.2432x over previous
import functools

import jax
import jax.numpy as jnp
from jax import lax
from jax.experimental import pallas as pl
from jax.experimental.pallas import tpu as pltpu
from jax.experimental.pallas import tpu_sc as plsc

N_VARS = 26
B = 1024
VOCAB = 1000
D = 64
H = 128

_TOT = N_VARS * B
_CHUNK = 128


def _sc_gather(tables_flat, flat_idx):
    info = plsc.get_sparse_core_info()
    nw = info.num_cores * info.num_subcores
    rows_per = _TOT // nw
    n_ch = -(-rows_per // _CHUNK)
    pad = n_ch * _CHUNK - rows_per
    idx3d = jnp.pad(flat_idx.reshape(nw, rows_per),
                    ((0, 0), (0, pad))).reshape(nw, n_ch, _CHUNK)

    mesh = plsc.VectorSubcoreMesh(core_axis_name="c", subcore_axis_name="s")

    @functools.partial(
        pl.kernel,
        mesh=mesh,
        out_type=jax.ShapeDtypeStruct((_TOT, D), jnp.float32),
        scratch_types=[
            pltpu.VMEM((n_ch, _CHUNK), jnp.int32),
            pltpu.VMEM((n_ch * _CHUNK, D), jnp.float32),
            pltpu.SemaphoreType.DMA,
        ],
        compiler_params=pltpu.CompilerParams(use_tc_tiling_on_sc=False),
    )
    def k(tab_hbm, idx_hbm, out_hbm, idx_v, rows_v, sem):
        wid = lax.axis_index("s") * info.num_cores + lax.axis_index("c")
        pltpu.sync_copy(idx_hbm.at[wid], idx_v)
        descs = [
            pltpu.async_copy(
                tab_hbm.at[idx_v.at[c]],
                rows_v.at[pl.ds(c * _CHUNK, _CHUNK)],
                sem,
            )
            for c in range(n_ch)
        ]
        for dsc in descs:
            dsc.wait()
        pltpu.sync_copy(rows_v.at[pl.ds(0, rows_per)],
                        out_hbm.at[pl.ds(wid * rows_per, rows_per)])

    return k(tables_flat, idx3d)


def _tc_body(emb_ref, w1_ref, b1_ref, w2_ref, b2_ref, wh_ref, bh_ref,
             oemb_ref, olog_ref):
    n = pl.program_id(0)

    @pl.when(n == 0)
    def _():
        acc = jnp.zeros((B, H), jnp.float32)
        for i in range(N_VARS):
            acc = acc + jnp.dot(emb_ref[i], w1_ref[i],
                                preferred_element_type=jnp.float32)
        h = jnp.maximum(acc + b1_ref[...], 0.0)
        oemb_ref[...] = (jnp.dot(h, w2_ref[...],
                                 preferred_element_type=jnp.float32)
                         + b2_ref[...])

    e = oemb_ref[...]
    olog_ref[0] = (jnp.dot(e, wh_ref[0], preferred_element_type=jnp.float32)
                   + bh_ref[0])


def _tc_mlp_heads(emb3, w1r, b1r, w2, b2r, wh, bhr):
    return pl.pallas_call(
        _tc_body,
        grid=(N_VARS,),
        in_specs=[
            pl.BlockSpec((N_VARS, B, D), lambda n: (0, 0, 0)),
            pl.BlockSpec((N_VARS, D, H), lambda n: (0, 0, 0)),
            pl.BlockSpec((1, H), lambda n: (0, 0)),
            pl.BlockSpec((H, D), lambda n: (0, 0)),
            pl.BlockSpec((1, D), lambda n: (0, 0)),
            pl.BlockSpec((1, D, VOCAB), lambda n: (n, 0, 0)),
            pl.BlockSpec((1, 1, VOCAB), lambda n: (n, 0, 0)),
        ],
        out_specs=[
            pl.BlockSpec((B, D), lambda n: (0, 0)),
            pl.BlockSpec((1, B, VOCAB), lambda n: (n, 0, 0)),
        ],
        out_shape=[
            jax.ShapeDtypeStruct((B, D), jnp.float32),
            jax.ShapeDtypeStruct((N_VARS, B, VOCAB), jnp.float32),
        ],
    )(emb3, w1r, b1r, w2, b2r, wh, bhr)


def kernel(context_vars, tables, W1, b1, W2, b2, Wh, bh):
    tables_flat = tables.reshape(N_VARS * VOCAB, D)
    flat_idx = (context_vars.astype(jnp.int32)
                + (jnp.arange(N_VARS, dtype=jnp.int32) * VOCAB)[:, None]
                ).reshape(_TOT)
    rows = _sc_gather(tables_flat, flat_idx)
    emb3 = rows.reshape(N_VARS, B, D)
    w1r = W1.reshape(N_VARS, D, H)
    embedding, logits = _tc_mlp_heads(
        emb3, w1r, b1.reshape(1, H), W2, b2.reshape(1, D),
        Wh, bh.reshape(N_VARS, 1, VOCAB))
    return embedding, logits


# speedup vs baseline: 5.7773x; 2.5755x over previous
import functools

import jax
import jax.numpy as jnp
from jax import lax
from jax.experimental import pallas as pl
from jax.experimental.pallas import tpu as pltpu
from jax.experimental.pallas import tpu_sc as plsc

N_VARS = 26
B = 1024
VOCAB = 1000
D = 64
H = 128

_TOT = N_VARS * B
_CHUNK = 128


def _sc_gather(tables_flat, loc_idx):
    info = plsc.get_sparse_core_info()
    nc, ns = info.num_cores, info.num_subcores
    vars_per_core = N_VARS // nc
    half_rows = vars_per_core * VOCAB

    mesh = plsc.VectorSubcoreMesh(core_axis_name="c", subcore_axis_name="s")

    @functools.partial(
        pl.kernel,
        mesh=mesh,
        out_type=jax.ShapeDtypeStruct((N_VARS // 2, B, 2 * D), jnp.float32),
        scratch_types=[
            pltpu.VMEM((1, B), jnp.int32),
            pltpu.VMEM((B, D), jnp.float32),
            pltpu.VMEM_SHARED((half_rows, D), jnp.float32),
            pltpu.SemaphoreType.DMA,
        ],
        compiler_params=pltpu.CompilerParams(use_tc_tiling_on_sc=False),
    )
    def k(tab_hbm, idx_hbm, out_hbm, idx_v, rows_v, tab_sp, sem):
        sid = lax.axis_index("s")
        cid = lax.axis_index("c")
        @pl.when(sid < vars_per_core)
        def _():
            pltpu.sync_copy(
                tab_hbm.at[pl.ds((cid * vars_per_core + sid) * VOCAB, VOCAB)],
                tab_sp.at[pl.ds(sid * VOCAB, VOCAB)])
            pltpu.sync_copy(idx_hbm.at[cid * vars_per_core + sid], idx_v)
            pltpu.async_copy(tab_sp.at[idx_v.at[0]], rows_v, sem).wait()
            v = cid * vars_per_core + sid
            pltpu.sync_copy(rows_v,
                            out_hbm.at[v // 2, :, pl.ds((v % 2) * D, D)])

    return k(tables_flat, loc_idx)


def _tc_body(emb_ref, w1_ref, b1_ref, w2_ref, b2_ref, wh_ref, bh_ref,
             oemb_ref, olog_ref):
    n = pl.program_id(0)

    @pl.when(n == 0)
    def _():
        acc = jnp.zeros((B, H), jnp.float32)
        for i in range(N_VARS // 2):
            acc = acc + jnp.dot(emb_ref[i], w1_ref[i],
                                preferred_element_type=jnp.float32)
        h = jnp.maximum(acc + b1_ref[...], 0.0)
        e = (jnp.dot(h, w2_ref[...], preferred_element_type=jnp.float32)
             + b2_ref[...])
        eye = (lax.broadcasted_iota(jnp.int32, (D, D), 0)
               == lax.broadcasted_iota(jnp.int32, (D, D), 1)
               ).astype(jnp.float32)
        oemb_ref[pl.ds(0, D), :] = lax.dot_general(
            eye, e, (((1,), (1,)), ((), ())),
            preferred_element_type=jnp.float32)
        oemb_ref[pl.ds(D, 1), :] = jnp.ones((1, B), jnp.float32)

    et_aug = oemb_ref[...]
    whb = jnp.concatenate([wh_ref[0], bh_ref[0]], axis=0)
    olog_ref[0] = lax.dot_general(whb, et_aug, (((0,), (0,)), ((), ())),
                                  preferred_element_type=jnp.float32)


def _tc_mlp_heads(emb3, w1r, b1r, w2, b2r, wh, bh3):
    return pl.pallas_call(
        _tc_body,
        grid=(N_VARS,),
        in_specs=[
            pl.BlockSpec((N_VARS // 2, B, 2 * D), lambda n: (0, 0, 0)),
            pl.BlockSpec((N_VARS // 2, 2 * D, H), lambda n: (0, 0, 0)),
            pl.BlockSpec((1, H), lambda n: (0, 0)),
            pl.BlockSpec((H, D), lambda n: (0, 0)),
            pl.BlockSpec((1, D), lambda n: (0, 0)),
            pl.BlockSpec((1, D, VOCAB), lambda n: (n, 0, 0)),
            pl.BlockSpec((1, 1, VOCAB), lambda n: (n, 0, 0)),
        ],
        out_specs=[
            pl.BlockSpec((D + 1, B), lambda n: (0, 0)),
            pl.BlockSpec((1, VOCAB, B), lambda n: (n, 0, 0)),
        ],
        out_shape=[
            jax.ShapeDtypeStruct((D + 1, B), jnp.float32),
            jax.ShapeDtypeStruct((N_VARS, VOCAB, B), jnp.float32),
        ],
    )(emb3, w1r, b1r, w2, b2r, wh, bh3)


def kernel(context_vars, tables, W1, b1, W2, b2, Wh, bh):
    tables_flat = tables.reshape(N_VARS * VOCAB, D)
    loc_idx = (context_vars.astype(jnp.int32)
               + ((jnp.arange(N_VARS, dtype=jnp.int32) % (N_VARS // 2))
                  * VOCAB)[:, None]).reshape(N_VARS, 1, B)
    emb_p = _sc_gather(tables_flat, loc_idx)
    w1r = W1.reshape(N_VARS // 2, 2 * D, H)
    emb_aug_t, logits_t = _tc_mlp_heads(
        emb_p, w1r, b1.reshape(1, H), W2, b2.reshape(1, D),
        Wh, bh.reshape(N_VARS, 1, VOCAB))
    return emb_aug_t[:D].T, jnp.transpose(logits_t, (0, 2, 1))
